# baseline (speedup 1.0000x reference)
import functools

import jax
import jax.numpy as jnp
from jax import lax
from jax.experimental import pallas as pl
from jax.experimental.pallas import tpu as pltpu
from jax.experimental.pallas import tpu_sc as plsc


_ROWS_PER_WORKER = 8


def _flip_table_on_sc(pos_emb):
    L, F = pos_emb.shape
    n_active = L // _ROWS_PER_WORKER
    mesh = plsc.VectorSubcoreMesh(core_axis_name="c", subcore_axis_name="s")

    @functools.partial(
        pl.kernel,
        out_type=jax.ShapeDtypeStruct((L, F), pos_emb.dtype),
        mesh=mesh,
        scratch_types=[pltpu.VMEM((1, F), pos_emb.dtype)],
    )
    def flip_kernel(pos_hbm, out_hbm, row_v):
        wid = lax.axis_index("s") * mesh.num_cores + lax.axis_index("c")

        @pl.when(wid < n_active)
        def _():
            base = wid * _ROWS_PER_WORKER
            for i in range(_ROWS_PER_WORKER):
                r = base + i
                pltpu.sync_copy(pos_hbm.at[pl.ds(L - 1 - r, 1)], row_v)
                pltpu.sync_copy(row_v, out_hbm.at[pl.ds(r, 1)])

    return flip_kernel(pos_emb)


def _add_body(s_ref, pf_ref, o_ref):
    o_ref[...] = s_ref[...] + pf_ref[...][None, :, :]


def kernel(sessions, pos_emb):
    B, L, F = sessions.shape
    flipped = _flip_table_on_sc(pos_emb)
    BLK_B = 128
    return pl.pallas_call(
        _add_body,
        grid=(B // BLK_B,),
        in_specs=[
            pl.BlockSpec((BLK_B, L, F), lambda i: (i, 0, 0)),
            pl.BlockSpec((L, F), lambda i: (0, 0)),
        ],
        out_specs=pl.BlockSpec((BLK_B, L, F), lambda i: (i, 0, 0)),
        out_shape=jax.ShapeDtypeStruct((B, L, F), sessions.dtype),
    )(sessions, flipped)


# speedup vs baseline: 1.0082x; 1.0082x over previous
import functools

import jax
import jax.numpy as jnp
from jax import lax
from jax.experimental import pallas as pl
from jax.experimental.pallas import tpu as pltpu
from jax.experimental.pallas import tpu_sc as plsc


_ROWS_PER_WORKER = 8


def _flip_table_on_sc(pos_emb):
    L, F = pos_emb.shape
    n_active = L // _ROWS_PER_WORKER
    mesh = plsc.VectorSubcoreMesh(core_axis_name="c", subcore_axis_name="s")

    @functools.partial(
        pl.kernel,
        out_type=jax.ShapeDtypeStruct((L, F), pos_emb.dtype),
        mesh=mesh,
        scratch_types=[
            pltpu.VMEM((_ROWS_PER_WORKER, F), pos_emb.dtype),
            pltpu.SemaphoreType.DMA,
        ],
    )
    def flip_kernel(pos_hbm, out_hbm, buf_v, sem):
        wid = lax.axis_index("s") * mesh.num_cores + lax.axis_index("c")

        @pl.when(wid < n_active)
        def _():
            base = wid * _ROWS_PER_WORKER
            handles = [
                pltpu.async_copy(
                    pos_hbm.at[pl.ds(L - 1 - base - i, 1)],
                    buf_v.at[pl.ds(i, 1)],
                    sem,
                )
                for i in range(_ROWS_PER_WORKER)
            ]
            for h in handles:
                h.wait()
            pltpu.sync_copy(buf_v, out_hbm.at[pl.ds(base, _ROWS_PER_WORKER)])

    return flip_kernel(pos_emb)


def _add_body(s_ref, pf_ref, o_ref):
    o_ref[...] = s_ref[...] + pf_ref[...][None, :, :]


def kernel(sessions, pos_emb):
    B, L, F = sessions.shape
    flipped = _flip_table_on_sc(pos_emb)
    BLK_B = 128
    return pl.pallas_call(
        _add_body,
        grid=(B // BLK_B,),
        in_specs=[
            pl.BlockSpec((BLK_B, L, F), lambda i: (i, 0, 0)),
            pl.BlockSpec((L, F), lambda i: (0, 0)),
        ],
        out_specs=pl.BlockSpec((BLK_B, L, F), lambda i: (i, 0, 0)),
        out_shape=jax.ShapeDtypeStruct((B, L, F), sessions.dtype),
    )(sessions, flipped)
